# baseline (speedup 1.0000x reference)
import functools

import jax
import jax.numpy as jnp
from jax import lax
from jax.experimental import pallas as pl
from jax.experimental.pallas import tpu as pltpu
from jax.experimental.pallas import tpu_sc as plsc

_LANES = 16
_CH = 4096


@functools.lru_cache(maxsize=None)
def _make_scatter_kernel(M, D, B, group):
  mesh = plsc.VectorSubcoreMesh(core_axis_name="c", subcore_axis_name="s")
  nc, ns = mesh.num_cores, mesh.num_subcores
  nw = nc * ns
  cols_per_w = D // nw
  n_ch = B // _CH
  n_groups = _CH // _LANES // group

  @functools.partial(
      pl.kernel,
      out_type=jax.ShapeDtypeStruct((D, M), jnp.float32),
      mesh=mesh,
      scratch_types=[
          pltpu.VMEM((M,), jnp.float32),
          pltpu.VMEM((_CH,), jnp.int32),
          pltpu.VMEM((_CH,), jnp.int32),
          pltpu.VMEM((_CH,), jnp.float32),
          pltpu.VMEM((_CH,), jnp.float32),
          pltpu.SemaphoreType.DMA,
          pltpu.SemaphoreType.DMA,
          pltpu.SemaphoreType.DMA,
          pltpu.SemaphoreType.DMA,
      ],
      compiler_params=pltpu.CompilerParams(needs_layout_passes=False),
  )
  def scatter_kernel(inpT, idxT, srcT, outT, colbuf, ib0, ib1, sb0, sb1,
                     scl, scs, sio0, sio1):
    wid = lax.axis_index("s") * nc + lax.axis_index("c")
    ibufs = (ib0, ib1)
    sbufs = (sb0, sb1)
    sios = (sio0, sio1)

    def col_j(c):
      return wid * cols_per_w + c

    def load_desc(c):
      return pltpu.make_async_copy(inpT.at[col_j(c)], colbuf, scl)

    def store_desc(c):
      return pltpu.make_async_copy(colbuf, outT.at[col_j(c)], scs)

    def io_descs(c, ch):
      b = ch % 2
      sl = pl.ds(ch * _CH, _CH)
      return (pltpu.make_async_copy(idxT.at[col_j(c), sl], ibufs[b], sios[b]),
              pltpu.make_async_copy(srcT.at[col_j(c), sl], sbufs[b], sios[b]))

    def start_io(c, ch):
      di, dsv = io_descs(c, ch)
      di.start()
      dsv.start()

    def wait_io(c, ch):
      di, dsv = io_descs(c, ch)
      di.wait()
      dsv.wait()

    load_desc(0).start()
    start_io(0, 0)

    for c in range(cols_per_w):
      load_desc(c).wait()
      for ch in range(n_ch):
        if ch + 1 < n_ch:
          start_io(c, ch + 1)
        elif c + 1 < cols_per_w:
          start_io(c + 1, 0)
        wait_io(c, ch)
        ibuf = ibufs[ch % 2]
        sbuf = sbufs[ch % 2]

        def chunk_body(t, carry, *, _ibuf=ibuf, _sbuf=sbuf):
          base = t * (group * _LANES)
          ent = []
          for k in range(group):
            off = base + k * _LANES
            idxv = _ibuf[pl.ds(off, _LANES)]
            srcv = _sbuf[pl.ds(off, _LANES)]
            _, keep = plsc.scan_count(idxv)
            ent.append((idxv, srcv, keep))
          for a, s, m in ent:
            plsc.store_scatter(colbuf, [a], s, mask=m)
          return carry

        lax.fori_loop(0, n_groups, chunk_body, 0)

      store_desc(c).start()
      if c + 1 < cols_per_w:
        store_desc(c).wait()
        load_desc(c + 1).start()

    store_desc(cols_per_w - 1).wait()

  return scatter_kernel


def _tc_transpose_half(x, h, bm=512):
  M, Dfull = x.shape
  Dh = Dfull // 2

  def body(x_ref, o_ref):
    blk = x_ref[:, h * Dh:(h + 1) * Dh]
    eye = jnp.eye(bm, dtype=jnp.float32)
    o_ref[...] = jax.lax.dot_general(
        blk, eye, (((0,), (0,)), ((), ())),
        preferred_element_type=jnp.float32)

  return pl.pallas_call(
      body,
      grid=(pl.cdiv(M, bm),),
      in_specs=[pl.BlockSpec((bm, Dfull), lambda i: (i, 0))],
      out_specs=pl.BlockSpec((Dh, bm), lambda i: (0, i)),
      out_shape=jax.ShapeDtypeStruct((Dh, M), x.dtype),
  )(x)


def _tc_merge(a, b, bm=512):
  Dh, M = a.shape

  def body(a_ref, b_ref, o_ref):
    eye = jnp.eye(bm, dtype=jnp.float32)
    at = jax.lax.dot_general(
        eye, a_ref[...], (((1,), (1,)), ((), ())),
        preferred_element_type=jnp.float32)
    bt = jax.lax.dot_general(
        eye, b_ref[...], (((1,), (1,)), ((), ())),
        preferred_element_type=jnp.float32)
    o_ref[...] = jnp.concatenate([at, bt], axis=1)

  return pl.pallas_call(
      body,
      grid=(pl.cdiv(M, bm),),
      in_specs=[pl.BlockSpec((Dh, bm), lambda i: (0, i)),
                pl.BlockSpec((Dh, bm), lambda i: (0, i))],
      out_specs=pl.BlockSpec((bm, 2 * Dh), lambda i: (i, 0)),
      out_shape=jax.ShapeDtypeStruct((M, 2 * Dh), a.dtype),
  )(a, b)


def kernel(input, dim, index, src):
  M, D = input.shape
  B = index.shape[0]
  Dh = D // 2
  idx = index + jnp.asarray(dim, index.dtype)
  f = _make_scatter_kernel(M, Dh, B, 8)
  outs = []
  for h in range(2):
    inT = _tc_transpose_half(input, h)
    idxT = idx[:, h * Dh:(h + 1) * Dh].T
    srcT = src[:, h * Dh:(h + 1) * Dh].T
    outs.append(f(inT, idxT, srcT))
  return _tc_merge(outs[0], outs[1])


# speedup vs baseline: 2.9152x; 2.9152x over previous
import functools

import jax
import jax.numpy as jnp
from jax import lax
from jax.experimental import pallas as pl
from jax.experimental.pallas import tpu as pltpu
from jax.experimental.pallas import tpu_sc as plsc

_LANES = 16
_CH = 4096


@functools.lru_cache(maxsize=None)
def _make_scatter_kernel(M, D, B, group):
  mesh = plsc.VectorSubcoreMesh(core_axis_name="c", subcore_axis_name="s")
  nc, ns = mesh.num_cores, mesh.num_subcores
  nw = nc * ns
  cols_per_w = D // nw
  n_ch = B // _CH
  n_groups = _CH // _LANES // group

  @functools.partial(
      pl.kernel,
      out_type=jax.ShapeDtypeStruct((D, M), jnp.float32),
      mesh=mesh,
      scratch_types=[
          pltpu.VMEM((M,), jnp.float32),
          pltpu.VMEM((_CH,), jnp.int32),
          pltpu.VMEM((_CH,), jnp.int32),
          pltpu.VMEM((_CH,), jnp.float32),
          pltpu.VMEM((_CH,), jnp.float32),
          pltpu.SemaphoreType.DMA,
          pltpu.SemaphoreType.DMA,
          pltpu.SemaphoreType.DMA,
          pltpu.SemaphoreType.DMA,
      ],
      compiler_params=pltpu.CompilerParams(needs_layout_passes=False),
  )
  def scatter_kernel(inpT, idxT, srcT, outT, colbuf, ib0, ib1, sb0, sb1,
                     scl, scs, sio0, sio1):
    wid = lax.axis_index("s") * nc + lax.axis_index("c")
    ibufs = (ib0, ib1)
    sbufs = (sb0, sb1)
    sios = (sio0, sio1)

    def col_j(c):
      return wid * cols_per_w + c

    def load_desc(c):
      return pltpu.make_async_copy(inpT.at[col_j(c)], colbuf, scl)

    def store_desc(c):
      return pltpu.make_async_copy(colbuf, outT.at[col_j(c)], scs)

    def io_descs(c, ch):
      b = ch % 2
      sl = pl.ds(ch * _CH, _CH)
      return (pltpu.make_async_copy(idxT.at[col_j(c), sl], ibufs[b], sios[b]),
              pltpu.make_async_copy(srcT.at[col_j(c), sl], sbufs[b], sios[b]))

    def start_io(c, ch):
      di, dsv = io_descs(c, ch)
      di.start()
      dsv.start()

    def wait_io(c, ch):
      di, dsv = io_descs(c, ch)
      di.wait()
      dsv.wait()

    load_desc(0).start()
    start_io(0, 0)

    for c in range(cols_per_w):
      load_desc(c).wait()
      for ch in range(n_ch):
        if ch + 1 < n_ch:
          start_io(c, ch + 1)
        elif c + 1 < cols_per_w:
          start_io(c + 1, 0)
        wait_io(c, ch)
        ibuf = ibufs[ch % 2]
        sbuf = sbufs[ch % 2]

        def chunk_body(t, carry, *, _ibuf=ibuf, _sbuf=sbuf):
          base = t * (group * _LANES)
          ent = []
          for k in range(group):
            off = base + k * _LANES
            idxv = _ibuf[pl.ds(off, _LANES)]
            srcv = _sbuf[pl.ds(off, _LANES)]
            _, keep = plsc.scan_count(idxv)
            ent.append((idxv, srcv, keep))
          for a, s, m in ent:
            plsc.store_scatter(colbuf, [a], s, mask=m)
          return carry

        lax.fori_loop(0, n_groups, chunk_body, 0)

      store_desc(c).start()
      if c + 1 < cols_per_w:
        store_desc(c).wait()
        load_desc(c + 1).start()

    store_desc(cols_per_w - 1).wait()

  return scatter_kernel


def kernel(input, dim, index, src):
  M, D = input.shape
  B = index.shape[0]
  idx = index + jnp.asarray(dim, index.dtype)
  f = _make_scatter_kernel(M, D, B, 16)
  outT = f(input.T, idx.T, src.T)
  return outT.T
